# baseline (device time: 29168 ns/iter reference)
import jax
import jax.numpy as jnp
from jax import lax
from jax.experimental import pallas as pl
from jax.experimental.pallas import tpu as pltpu

M = 1024
N = 512
C = 128
MAX_CHUNKS = M // C


def kernel(x, dest):
    d0 = dest == 0
    cz = jnp.cumsum(d0.astype(jnp.int32))
    c0 = cz[-1].astype(jnp.int32)
    i = jnp.arange(M, dtype=jnp.int32)
    p = jnp.where(d0, cz - 1, c0 + i - cz)
    order = jnp.zeros((M,), jnp.int32).at[p].set(
        i, unique_indices=True, mode="promise_in_bounds"
    )
    xs = jnp.take(x, order, axis=0)

    def body(c_ref, xs_ref, out_ref, stg_ref, send_sems, recv_sems):
        my_x = lax.axis_index("x")
        my_y = lax.axis_index("y")
        peer = (1 - my_x, my_y)

        c = c_ref[0]
        is0 = my_x == 0
        src_start = jnp.where(is0, c, 0)
        src_al = (src_start // 8) * 8
        n_send = jnp.where(is0, M - c, c)
        total8 = ((n_send + (src_start - src_al) + 7) // 8) * 8
        n_cs = (total8 + C - 1) // C
        peer_c0 = M - c
        peer_src_start = jnp.where(is0, 0, peer_c0)
        peer_lead = peer_src_start % 8
        peer_total8 = ((n_send + peer_lead + 7) // 8) * 8
        n_cr = (peer_total8 + C - 1) // C

        barrier_sem = pltpu.get_barrier_semaphore()
        pl.semaphore_signal(
            barrier_sem, inc=1, device_id=peer,
            device_id_type=pl.DeviceIdType.MESH,
        )
        pl.semaphore_wait(barrier_sem, 1)

        for j in range(MAX_CHUNKS):
            @pl.when(j < n_cs)
            def _(j=j):
                off = jnp.minimum(j * C, total8 - C)
                pltpu.make_async_remote_copy(
                    src_ref=xs_ref.at[pl.ds(src_al + off, C)],
                    dst_ref=stg_ref.at[pl.ds(off, C)],
                    send_sem=send_sems.at[j],
                    recv_sem=recv_sems.at[j],
                    device_id=peer,
                    device_id_type=pl.DeviceIdType.MESH,
                ).start()

        for j in range(MAX_CHUNKS):
            @pl.when(j < n_cs)
            def _(j=j):
                pltpu.make_async_remote_copy(
                    src_ref=xs_ref.at[pl.ds(0, C)],
                    dst_ref=stg_ref.at[pl.ds(0, C)],
                    send_sem=send_sems.at[j],
                    recv_sem=recv_sems.at[j],
                    device_id=peer,
                    device_id_type=pl.DeviceIdType.MESH,
                ).wait_send()
        for j in range(MAX_CHUNKS):
            @pl.when(j < n_cr)
            def _(j=j):
                pltpu.make_async_remote_copy(
                    src_ref=xs_ref.at[pl.ds(0, C)],
                    dst_ref=stg_ref.at[pl.ds(0, C)],
                    send_sem=send_sems.at[j],
                    recv_sem=recv_sems.at[j],
                    device_id=peer,
                    device_id_type=pl.DeviceIdType.MESH,
                ).wait_recv()

        lead = (M - c) % 8
        shift = jnp.where(is0, c, (M - lead) % M)
        rolled = pltpu.roll(stg_ref[...], shift, 0)
        idx = lax.broadcasted_iota(jnp.int32, (M, 1), 0)
        keep_mask = (idx < c) == is0
        out_ref[...] = jnp.where(keep_mask, xs_ref[...], rolled)

    out = pl.pallas_call(
        body,
        out_shape=jax.ShapeDtypeStruct((M, N), jnp.float32),
        in_specs=[
            pl.BlockSpec(memory_space=pltpu.SMEM),
            pl.BlockSpec(memory_space=pltpu.VMEM),
        ],
        out_specs=pl.BlockSpec(memory_space=pltpu.VMEM),
        scratch_shapes=[
            pltpu.VMEM((M, N), jnp.float32),
            pltpu.SemaphoreType.DMA((MAX_CHUNKS,)),
            pltpu.SemaphoreType.DMA((MAX_CHUNKS,)),
        ],
        compiler_params=pltpu.CompilerParams(collective_id=0),
    )(c0.reshape(1), xs)
    return out


# device time: 25875 ns/iter; 1.1273x vs baseline; 1.1273x over previous
import jax
import jax.numpy as jnp
from jax import lax
from jax.experimental import pallas as pl
from jax.experimental.pallas import tpu as pltpu

M = 1024
N = 512
C = 64
MAX_CHUNKS = M // C


def kernel(x, dest):
    d0 = dest == 0
    cz = jnp.cumsum(d0.astype(jnp.int32))
    c0 = cz[-1].astype(jnp.int32)
    i = jnp.arange(M, dtype=jnp.int32)
    p = jnp.where(d0, cz - 1, c0 + i - cz)
    xs = jnp.zeros_like(x).at[p].set(
        x, unique_indices=True, mode="promise_in_bounds"
    )

    def body(c_ref, xs_ref, out_ref, stg_ref,
             xsend_sems, xrecv_sems, ysend_sems, yrecv_sems):
        my_x = lax.axis_index("x")
        my_y = lax.axis_index("y")
        xpeer = (1 - my_x, my_y)
        ypeer = (my_x, 1 - my_y)

        c = c_ref[0]
        is0 = my_x == 0
        src_start = jnp.where(is0, c, 0)
        src_al = (src_start // 8) * 8
        n_send = jnp.where(is0, M - c, c)
        total8 = ((n_send + (src_start - src_al) + 7) // 8) * 8
        n_cs = (total8 + C - 1) // C
        peer_c0 = M - c
        peer_src_start = jnp.where(is0, 0, peer_c0)
        peer_lead = peer_src_start % 8
        peer_total8 = ((n_send + peer_lead + 7) // 8) * 8
        n_cr = (peer_total8 + C - 1) // C

        barrier_sem = pltpu.get_barrier_semaphore()
        for nbr in (xpeer, ypeer):
            pl.semaphore_signal(
                barrier_sem, inc=1, device_id=nbr,
                device_id_type=pl.DeviceIdType.MESH,
            )
        pl.semaphore_wait(barrier_sem, 2)

        for j in range(MAX_CHUNKS):
            @pl.when((j < n_cs) & ((j % 2) == my_y))
            def _(j=j):
                off = jnp.minimum(j * C, total8 - C)
                pltpu.make_async_remote_copy(
                    src_ref=xs_ref.at[pl.ds(src_al + off, C)],
                    dst_ref=stg_ref.at[pl.ds(off, C)],
                    send_sem=xsend_sems.at[j],
                    recv_sem=xrecv_sems.at[j],
                    device_id=xpeer,
                    device_id_type=pl.DeviceIdType.MESH,
                ).start()

        for j in range(MAX_CHUNKS):
            @pl.when((j < n_cr) & ((j % 2) == my_y))
            def _(j=j):
                pltpu.make_async_remote_copy(
                    src_ref=xs_ref.at[pl.ds(0, C)],
                    dst_ref=stg_ref.at[pl.ds(0, C)],
                    send_sem=xsend_sems.at[j],
                    recv_sem=xrecv_sems.at[j],
                    device_id=xpeer,
                    device_id_type=pl.DeviceIdType.MESH,
                ).wait_recv()
                off = jnp.minimum(j * C, peer_total8 - C)
                pltpu.make_async_remote_copy(
                    src_ref=stg_ref.at[pl.ds(off, C)],
                    dst_ref=stg_ref.at[pl.ds(off, C)],
                    send_sem=ysend_sems.at[j],
                    recv_sem=yrecv_sems.at[j],
                    device_id=ypeer,
                    device_id_type=pl.DeviceIdType.MESH,
                ).start()

        for j in range(MAX_CHUNKS):
            @pl.when((j < n_cr) & ((j % 2) != my_y))
            def _(j=j):
                pltpu.make_async_remote_copy(
                    src_ref=stg_ref.at[pl.ds(0, C)],
                    dst_ref=stg_ref.at[pl.ds(0, C)],
                    send_sem=ysend_sems.at[j],
                    recv_sem=yrecv_sems.at[j],
                    device_id=ypeer,
                    device_id_type=pl.DeviceIdType.MESH,
                ).wait_recv()

        for j in range(MAX_CHUNKS):
            @pl.when((j < n_cs) & ((j % 2) == my_y))
            def _(j=j):
                pltpu.make_async_remote_copy(
                    src_ref=xs_ref.at[pl.ds(0, C)],
                    dst_ref=stg_ref.at[pl.ds(0, C)],
                    send_sem=xsend_sems.at[j],
                    recv_sem=xrecv_sems.at[j],
                    device_id=xpeer,
                    device_id_type=pl.DeviceIdType.MESH,
                ).wait_send()
        for j in range(MAX_CHUNKS):
            @pl.when((j < n_cr) & ((j % 2) == my_y))
            def _(j=j):
                pltpu.make_async_remote_copy(
                    src_ref=stg_ref.at[pl.ds(0, C)],
                    dst_ref=stg_ref.at[pl.ds(0, C)],
                    send_sem=ysend_sems.at[j],
                    recv_sem=yrecv_sems.at[j],
                    device_id=ypeer,
                    device_id_type=pl.DeviceIdType.MESH,
                ).wait_send()

        lead = (M - c) % 8
        shift = jnp.where(is0, c, (M - lead) % M)
        rolled = pltpu.roll(stg_ref[...], shift, 0)
        idx = lax.broadcasted_iota(jnp.int32, (M, 1), 0)
        keep_mask = (idx < c) == is0
        out_ref[...] = jnp.where(keep_mask, xs_ref[...], rolled)

    out = pl.pallas_call(
        body,
        out_shape=jax.ShapeDtypeStruct((M, N), jnp.float32),
        in_specs=[
            pl.BlockSpec(memory_space=pltpu.SMEM),
            pl.BlockSpec(memory_space=pltpu.VMEM),
        ],
        out_specs=pl.BlockSpec(memory_space=pltpu.VMEM),
        scratch_shapes=[
            pltpu.VMEM((M, N), jnp.float32),
            pltpu.SemaphoreType.DMA((MAX_CHUNKS,)),
            pltpu.SemaphoreType.DMA((MAX_CHUNKS,)),
            pltpu.SemaphoreType.DMA((MAX_CHUNKS,)),
            pltpu.SemaphoreType.DMA((MAX_CHUNKS,)),
        ],
        compiler_params=pltpu.CompilerParams(collective_id=0),
    )(c0.reshape(1), xs)
    return out
